# baseline (speedup 1.0000x reference)
import functools

import jax
import jax.numpy as jnp
from jax import lax
from jax.experimental import pallas as pl
from jax.experimental.pallas import tpu as pltpu
from jax.experimental.pallas import tpu_sc as plsc

N_ROWS = 1048576
NW = 32
ROWS_W = N_ROWS // NW
CH = 4096
NCH = ROWS_W // CH
L = 16


def _sc_rejection(eps_flat, prob, consts):
    mesh = plsc.VectorSubcoreMesh(core_axis_name="c", subcore_axis_name="s")

    @functools.partial(
        pl.kernel,
        mesh=mesh,
        compiler_params=pltpu.CompilerParams(needs_layout_passes=False),
        out_type=jax.ShapeDtypeStruct((2 * N_ROWS,), jnp.float32),
        scratch_types=[
            pltpu.VMEM((2 * CH,), jnp.float32),
            pltpu.VMEM((CH,), jnp.float32),
            pltpu.VMEM((4 * L,), jnp.float32),
            pltpu.SemaphoreType.DMA,
        ],
    )
    def k(eps_hbm, prob_hbm, consts_hbm, out_hbm, ebuf, pbuf, cbuf, sem):
        cid = lax.axis_index("c")
        sid = lax.axis_index("s")
        wid = sid * 2 + cid
        pltpu.sync_copy(consts_hbm, cbuf)
        s0 = cbuf[pl.ds(0, L)]
        s1 = cbuf[pl.ds(L, L)]
        t0 = cbuf[pl.ds(2 * L, L)]
        t1 = cbuf[pl.ds(3 * L, L)]
        idx2 = lax.iota(jnp.int32, L) * 2
        row0 = wid * ROWS_W

        def chunk_body(ci, carry):
            rbase = row0 + ci * CH
            pltpu.sync_copy(eps_hbm.at[pl.ds(rbase * 2, 2 * CH)], ebuf)
            pltpu.sync_copy(prob_hbm.at[pl.ds(rbase, CH)], pbuf)

            def body(j, c2):
                ie = idx2 + j * 32
                io = ie + 1
                e0 = plsc.load_gather(ebuf, [ie])
                e1 = plsc.load_gather(ebuf, [io])
                z0 = e0 * s0 + t0
                z1 = e1 * s1 + t1
                tot = z0 * z0 + z1 * z1
                p = pbuf[pl.ds(j * L, L)]
                acc = jnp.exp(tot * -0.5) > p
                zero = jnp.zeros((L,), jnp.float32)
                plsc.store_scatter(ebuf, [ie], jnp.where(acc, z0, zero))
                plsc.store_scatter(ebuf, [io], jnp.where(acc, z1, zero))
                return c2

            lax.fori_loop(0, CH // L, body, 0)
            pltpu.sync_copy(ebuf, out_hbm.at[pl.ds(rbase * 2, 2 * CH)])
            return carry

        lax.fori_loop(0, NCH, chunk_body, 0)

    return k(eps_flat, prob, consts)


def kernel(eps, prob, prop_scale, prop_shift):
    consts = jnp.concatenate([
        jnp.broadcast_to(prop_scale[0], (L,)),
        jnp.broadcast_to(prop_scale[1], (L,)),
        jnp.broadcast_to(prop_shift[0], (L,)),
        jnp.broadcast_to(prop_shift[1], (L,)),
    ]).astype(jnp.float32)
    out_flat = _sc_rejection(eps.reshape(-1), prob, consts)
    return out_flat.reshape(N_ROWS, 2)
